# speedup vs baseline: 1.1324x; 1.1324x over previous
import functools

import jax
import jax.numpy as jnp
from jax.experimental import pallas as pl
from jax.experimental.pallas import tpu as pltpu

N_NODES = 10000
D = 128
ROW_BLK = 2000


def _matmul_body(h_ref, wt_ref, b_ref, wa_ref, z_ref, a_ref):
    z = jnp.dot(h_ref[...], wt_ref[...], preferred_element_type=jnp.float32)
    z = z + b_ref[...]
    z_ref[...] = z
    a_ref[...] = jnp.dot(z, wa_ref[...], preferred_element_type=jnp.float32)


def _fused_matmul(h, W_fc, b_fc, W_a):
    wt = W_fc.T
    wa = W_a.reshape(2, D).T
    grid = N_NODES // ROW_BLK
    z, a = pl.pallas_call(
        _matmul_body,
        grid=(grid,),
        in_specs=[
            pl.BlockSpec((ROW_BLK, D), lambda i: (i, 0)),
            pl.BlockSpec((D, D), lambda i: (0, 0)),
            pl.BlockSpec((D,), lambda i: (0,)),
            pl.BlockSpec((D, 2), lambda i: (0, 0)),
        ],
        out_specs=[
            pl.BlockSpec((ROW_BLK, D), lambda i: (i, 0)),
            pl.BlockSpec((ROW_BLK, 2), lambda i: (i, 0)),
        ],
        out_shape=[
            jax.ShapeDtypeStruct((N_NODES, D), jnp.float32),
            jax.ShapeDtypeStruct((N_NODES, 2), jnp.float32),
        ],
    )(h, wt, b_fc, wa)
    return z, a


def kernel(h, edge_index, W_fc, b_fc, W_a, b_a):
    z, a = _fused_matmul(h, W_fc, b_fc, W_a)
    src = edge_index[0].astype(jnp.int32)
    dst = edge_index[1].astype(jnp.int32)
    a1 = a[:, 0]
    a2 = a[:, 1]
    e = a1[src] + a2[dst] + b_a[0]
    e = jnp.where(e >= 0, e, 0.01 * e)
    ex = jnp.exp(e)
    denom = jax.ops.segment_sum(ex, dst, num_segments=N_NODES)
    alpha = ex / denom[dst]
    h_out = jax.ops.segment_sum(alpha[:, None] * z[src], dst, num_segments=N_NODES)
    return h_out


# speedup vs baseline: 9.7607x; 8.6199x over previous
import functools

import jax
import jax.numpy as jnp
from jax import lax
from jax.experimental import pallas as pl
from jax.experimental.pallas import tpu as pltpu
from jax.experimental.pallas import tpu_sc as plsc

N = 10000
NP = 10240
D = 128
E = 320000
NC = 2
NS = 16
NW = NC * NS
EPW = 10240
CHUNKS = 80
CL = 128
EP = NW * EPW
RPT = NP // NS
ROW_BLK = 2000


def _matmul_body(h_ref, wt_ref, b_ref, wa_ref, ba_ref, z_ref, a_ref):
    z = jnp.dot(h_ref[...], wt_ref[...], preferred_element_type=jnp.float32)
    z = z + b_ref[...]
    z_ref[...] = z
    a_ref[...] = (
        jnp.dot(z, wa_ref[...], preferred_element_type=jnp.float32) + ba_ref[...]
    )


def _fused_matmul(h, W_fc, b_fc, W_a, b_a):
    wt = W_fc.T
    wa = W_a.reshape(2, D).T
    ba2 = jnp.concatenate([jnp.zeros((1,), jnp.float32), b_a])
    grid = N // ROW_BLK
    z, a = pl.pallas_call(
        _matmul_body,
        grid=(grid,),
        in_specs=[
            pl.BlockSpec((ROW_BLK, D), lambda i: (i, 0)),
            pl.BlockSpec((D, D), lambda i: (0, 0)),
            pl.BlockSpec((D,), lambda i: (0,)),
            pl.BlockSpec((D, 2), lambda i: (0, 0)),
            pl.BlockSpec((2,), lambda i: (0,)),
        ],
        out_specs=[
            pl.BlockSpec((ROW_BLK, D), lambda i: (i, 0)),
            pl.BlockSpec((ROW_BLK, 2), lambda i: (i, 0)),
        ],
        out_shape=[
            jax.ShapeDtypeStruct((NP, D), jnp.float32),
            jax.ShapeDtypeStruct((NP, 2), jnp.float32),
        ],
    )(h, wt, b_fc, wa, ba2)
    return z, a


def _combine_body(p0_ref, p1_ref, o_ref):
    o_ref[...] = p0_ref[...] + p1_ref[...]


def _den_add_body(d_ref, o_ref):
    o_ref[...] = d_ref[0] + d_ref[1]


def _den_add(den):
    out = pl.pallas_call(
        _den_add_body,
        out_shape=jax.ShapeDtypeStruct((NP // D, D), jnp.float32),
    )(den.reshape(NC, NP // D, D))
    return out.reshape(NP)


def _combine(p0, p1):
    return pl.pallas_call(
        _combine_body,
        grid=(N // ROW_BLK,),
        in_specs=[
            pl.BlockSpec((ROW_BLK, D), lambda i: (i, 0)),
            pl.BlockSpec((ROW_BLK, D), lambda i: (i, 0)),
        ],
        out_specs=pl.BlockSpec((ROW_BLK, D), lambda i: (i, 0)),
        out_shape=jax.ShapeDtypeStruct((N, D), jnp.float32),
    )(p0, p1)


_MESH = plsc.VectorSubcoreMesh(core_axis_name="c", subcore_axis_name="s")
_SC_PARAMS = pltpu.CompilerParams(needs_layout_passes=False)


@functools.partial(
    pl.kernel,
    out_type=[
        jax.ShapeDtypeStruct((NW, EPW), jnp.float32),
        jax.ShapeDtypeStruct((NC, NP), jnp.float32),
    ],
    mesh=_MESH,
    compiler_params=_SC_PARAMS,
    scratch_types=[
        pltpu.VMEM((NP,), jnp.float32),
        pltpu.VMEM((NP,), jnp.float32),
        pltpu.VMEM((CHUNKS, CL), jnp.int32),
        pltpu.VMEM((CHUNKS, CL), jnp.int32),
        pltpu.VMEM((EPW,), jnp.float32),
        pltpu.VMEM((RPT,), jnp.float32),
        pltpu.VMEM_SHARED((NP,), jnp.float32),
    ],
)
def _edge_pass_a(src_hbm, dst_hbm, a12_hbm, ex_hbm, den_hbm,
                 a1_v, a2_v, src_v, dst_v, ex_v, zero_v, den_sh):
    c = lax.axis_index("c")
    s = lax.axis_index("s")
    w = s * NC + c
    pltpu.sync_copy(a12_hbm.at[0], a1_v)
    pltpu.sync_copy(a12_hbm.at[1], a2_v)
    pltpu.sync_copy(src_hbm.at[w], src_v)
    pltpu.sync_copy(dst_hbm.at[w], dst_v)

    def zb(i, carry):
        zero_v[pl.ds(i * 16, 16)] = jnp.zeros((16,), jnp.float32)
        return carry

    lax.fori_loop(0, RPT // 16, zb, 0)
    pltpu.sync_copy(zero_v, den_sh.at[pl.ds(s * RPT, RPT)])
    plsc.subcore_barrier()

    iota16 = lax.iota(jnp.int32, 16)
    wbase = w * EPW

    def chunk_body(j, carry):
        for k in range(CL // 16):
            srcv = src_v[j, pl.ds(k * 16, 16)]
            dstv = dst_v[j, pl.ds(k * 16, 16)]
            a1 = plsc.load_gather(a1_v, [srcv])
            a2 = plsc.load_gather(a2_v, [dstv])
            e = a1 + a2
            e = jnp.where(e >= 0.0, e, 0.01 * e)
            flat = wbase + j * CL + k * 16 + iota16
            ex = jnp.where(flat < E, jnp.exp(e), 0.0)
            ex_v[pl.ds(j * CL + k * 16, 16)] = ex
        pltpu.sync_copy(ex_v.at[pl.ds(j * CL, CL)],
                        den_sh.at[dst_v.at[j]], add=True)
        return carry

    lax.fori_loop(0, CHUNKS, chunk_body, 0)
    plsc.subcore_barrier()
    pltpu.sync_copy(ex_v, ex_hbm.at[w])
    pltpu.sync_copy(den_sh.at[pl.ds(s * RPT, RPT)],
                    den_hbm.at[c, pl.ds(s * RPT, RPT)])


@functools.partial(
    pl.kernel,
    out_type=jax.ShapeDtypeStruct((NC, NP, D), jnp.float32),
    mesh=_MESH,
    compiler_params=_SC_PARAMS,
    scratch_types=[
        pltpu.VMEM((CL,), jnp.int32),
        pltpu.VMEM((CL,), jnp.int32),
        pltpu.VMEM((CL,), jnp.float32),
        pltpu.VMEM((CL,), jnp.float32),
        pltpu.VMEM((NP,), jnp.float32),
        pltpu.VMEM((CL, D), jnp.float32),
        pltpu.VMEM_SHARED((NP, D), jnp.float32),
        pltpu.SemaphoreType.DMA,
    ],
)
def _edge_pass_b(src_hbm, dst_hbm, ex_hbm, den_hbm, z_hbm, part_hbm,
                 src_c, dst_c, ex_c, al_c, den_v, rows_v, accum_sh, sem):
    c = lax.axis_index("c")
    s = lax.axis_index("s")
    w = s * NC + c
    pltpu.sync_copy(den_hbm, den_v)

    def zb(i, carry):
        r = i // 8
        kk = i % 8
        rows_v[r, pl.ds(kk * 16, 16)] = jnp.zeros((16,), jnp.float32)
        return carry

    lax.fori_loop(0, CL * 8, zb, 0)
    for t in range(RPT // CL):
        pltpu.sync_copy(rows_v, accum_sh.at[pl.ds(s * RPT + t * CL, CL)])
    plsc.subcore_barrier()

    def chunk_body(j, carry):
        pltpu.async_copy(src_hbm.at[w, j], src_c, sem).wait()
        pltpu.async_copy(dst_hbm.at[w, j], dst_c, sem).wait()
        pltpu.async_copy(ex_hbm.at[w, j], ex_c, sem).wait()
        pltpu.async_copy(z_hbm.at[src_c], rows_v, sem).wait()
        for k in range(CL // 16):
            dstv = dst_c[pl.ds(k * 16, 16)]
            den = plsc.load_gather(den_v, [dstv])
            exv = ex_c[pl.ds(k * 16, 16)]
            al_c[pl.ds(k * 16, 16)] = exv / jnp.maximum(den, 1e-38)

        def eb(e, cc):
            alv = plsc.load_gather(al_c, [jnp.full((16,), e, jnp.int32)])
            for kk in range(D // 16):
                seg = rows_v[e, pl.ds(kk * 16, 16)]
                rows_v[e, pl.ds(kk * 16, 16)] = seg * alv
            return cc

        lax.fori_loop(0, CL, eb, 0)
        pltpu.sync_copy(rows_v, accum_sh.at[dst_c], add=True)
        return carry

    lax.fori_loop(0, CHUNKS, chunk_body, 0)
    plsc.subcore_barrier()
    for t in range(RPT // CL):
        r0 = s * RPT + t * CL
        pltpu.sync_copy(accum_sh.at[pl.ds(r0, CL)],
                        part_hbm.at[c, pl.ds(r0, CL)])


def kernel(h, edge_index, W_fc, b_fc, W_a, b_a):
    z, a = _fused_matmul(h, W_fc, b_fc, W_a, b_a)
    a12 = a.T
    pad = EP - E
    src_p = jnp.concatenate(
        [edge_index[0].astype(jnp.int32), jnp.zeros((pad,), jnp.int32)]
    ).reshape(NW, CHUNKS, CL)
    dst_p = jnp.concatenate(
        [edge_index[1].astype(jnp.int32), jnp.zeros((pad,), jnp.int32)]
    ).reshape(NW, CHUNKS, CL)
    ex, den = _edge_pass_a(src_p, dst_p, a12)
    ex = ex.reshape(NW, CHUNKS, CL)
    part = _edge_pass_b(src_p, dst_p, ex, _den_add(den), z)
    return _combine(part[0], part[1])


# speedup vs baseline: 13.5915x; 1.3925x over previous
import functools

import jax
import jax.numpy as jnp
from jax import lax
from jax.experimental import pallas as pl
from jax.experimental.pallas import tpu as pltpu
from jax.experimental.pallas import tpu_sc as plsc

N = 10000
NP = 10240
D = 128
E = 320000
NC = 2
NS = 16
NW = NC * NS
EPW = 10240
CHUNKS = 80
CL = 128
EP = NW * EPW
RPT = NP // NS
ROW_BLK = 2000


def _matmul_body(h_ref, wt_ref, b_ref, wa_ref, ba_ref, z_ref, a_ref):
    z = jnp.dot(h_ref[...], wt_ref[...], preferred_element_type=jnp.float32)
    z = z + b_ref[...]
    z_ref[...] = z
    a_ref[...] = (
        jnp.dot(z, wa_ref[...], preferred_element_type=jnp.float32) + ba_ref[...]
    )


def _fused_matmul(h, W_fc, b_fc, W_a, b_a):
    wt = W_fc.T
    wa = W_a.reshape(2, D).T
    ba2 = jnp.concatenate([jnp.zeros((1,), jnp.float32), b_a])
    grid = N // ROW_BLK
    z, a = pl.pallas_call(
        _matmul_body,
        grid=(grid,),
        in_specs=[
            pl.BlockSpec((ROW_BLK, D), lambda i: (i, 0)),
            pl.BlockSpec((D, D), lambda i: (0, 0)),
            pl.BlockSpec((D,), lambda i: (0,)),
            pl.BlockSpec((D, 2), lambda i: (0, 0)),
            pl.BlockSpec((2,), lambda i: (0,)),
        ],
        out_specs=[
            pl.BlockSpec((ROW_BLK, D), lambda i: (i, 0)),
            pl.BlockSpec((ROW_BLK, 2), lambda i: (i, 0)),
        ],
        out_shape=[
            jax.ShapeDtypeStruct((NP, D), jnp.float32),
            jax.ShapeDtypeStruct((NP, 2), jnp.float32),
        ],
    )(h, wt, b_fc, wa, ba2)
    return z, a


def _combine_body(p0_ref, p1_ref, o_ref):
    o_ref[...] = p0_ref[...] + p1_ref[...]


def _den_add_body(d_ref, o_ref):
    o_ref[...] = d_ref[0] + d_ref[1]


def _den_add(den):
    out = pl.pallas_call(
        _den_add_body,
        out_shape=jax.ShapeDtypeStruct((NP // D, D), jnp.float32),
    )(den.reshape(NC, NP // D, D))
    return out.reshape(NP)


def _combine(p0, p1):
    return pl.pallas_call(
        _combine_body,
        grid=(N // ROW_BLK,),
        in_specs=[
            pl.BlockSpec((ROW_BLK, D), lambda i: (i, 0)),
            pl.BlockSpec((ROW_BLK, D), lambda i: (i, 0)),
        ],
        out_specs=pl.BlockSpec((ROW_BLK, D), lambda i: (i, 0)),
        out_shape=jax.ShapeDtypeStruct((N, D), jnp.float32),
    )(p0, p1)


_MESH = plsc.VectorSubcoreMesh(core_axis_name="c", subcore_axis_name="s")
_SC_PARAMS = pltpu.CompilerParams(needs_layout_passes=False)


@functools.partial(
    pl.kernel,
    out_type=[
        jax.ShapeDtypeStruct((NW, EPW), jnp.float32),
        jax.ShapeDtypeStruct((NC, NP), jnp.float32),
    ],
    mesh=_MESH,
    compiler_params=_SC_PARAMS,
    scratch_types=[
        pltpu.VMEM((NP,), jnp.float32),
        pltpu.VMEM((NP,), jnp.float32),
        pltpu.VMEM((CHUNKS, CL), jnp.int32),
        pltpu.VMEM((CHUNKS, CL), jnp.int32),
        pltpu.VMEM((EPW,), jnp.float32),
        pltpu.VMEM((RPT,), jnp.float32),
        pltpu.VMEM_SHARED((NP,), jnp.float32),
    ],
)
def _edge_pass_a(src_hbm, dst_hbm, a12_hbm, ex_hbm, den_hbm,
                 a1_v, a2_v, src_v, dst_v, ex_v, zero_v, den_sh):
    c = lax.axis_index("c")
    s = lax.axis_index("s")
    w = s * NC + c
    pltpu.sync_copy(a12_hbm.at[0], a1_v)
    pltpu.sync_copy(a12_hbm.at[1], a2_v)
    pltpu.sync_copy(src_hbm.at[w], src_v)
    pltpu.sync_copy(dst_hbm.at[w], dst_v)

    def zb(i, carry):
        zero_v[pl.ds(i * 16, 16)] = jnp.zeros((16,), jnp.float32)
        return carry

    lax.fori_loop(0, RPT // 16, zb, 0)
    pltpu.sync_copy(zero_v, den_sh.at[pl.ds(s * RPT, RPT)])
    plsc.subcore_barrier()

    iota16 = lax.iota(jnp.int32, 16)
    wbase = w * EPW

    def chunk_body(j, carry):
        for k in range(CL // 16):
            srcv = src_v[j, pl.ds(k * 16, 16)]
            dstv = dst_v[j, pl.ds(k * 16, 16)]
            a1 = plsc.load_gather(a1_v, [srcv])
            a2 = plsc.load_gather(a2_v, [dstv])
            e = a1 + a2
            e = jnp.where(e >= 0.0, e, 0.01 * e)
            flat = wbase + j * CL + k * 16 + iota16
            ex = jnp.where(flat < E, jnp.exp(e), 0.0)
            ex_v[pl.ds(j * CL + k * 16, 16)] = ex
        pltpu.sync_copy(ex_v.at[pl.ds(j * CL, CL)],
                        den_sh.at[dst_v.at[j]], add=True)
        return carry

    lax.fori_loop(0, CHUNKS, chunk_body, 0)
    plsc.subcore_barrier()
    pltpu.sync_copy(ex_v, ex_hbm.at[w])
    pltpu.sync_copy(den_sh.at[pl.ds(s * RPT, RPT)],
                    den_hbm.at[c, pl.ds(s * RPT, RPT)])


CLB = 64
NCHB = EPW // CLB
HALF = NCHB // 2


@functools.partial(
    pl.kernel,
    out_type=jax.ShapeDtypeStruct((NC, NP, D), jnp.float32),
    mesh=_MESH,
    compiler_params=_SC_PARAMS,
    scratch_types=[
        pltpu.VMEM((2, CLB), jnp.int32),
        pltpu.VMEM((2, CLB), jnp.int32),
        pltpu.VMEM((CLB,), jnp.float32),
        pltpu.VMEM((CLB,), jnp.float32),
        pltpu.VMEM((CLB,), jnp.float32),
        pltpu.VMEM((CLB,), jnp.float32),
        pltpu.VMEM((CLB,), jnp.int32),
        pltpu.VMEM((CLB,), jnp.int32),
        pltpu.VMEM((NP,), jnp.float32),
        pltpu.VMEM((CLB, D), jnp.float32),
        pltpu.VMEM((CLB, D), jnp.float32),
        pltpu.VMEM_SHARED((NP, D), jnp.float32),
        pltpu.SemaphoreType.DMA,
        pltpu.SemaphoreType.DMA,
        pltpu.SemaphoreType.DMA,
        pltpu.SemaphoreType.DMA,
        pltpu.SemaphoreType.DMA,
        pltpu.SemaphoreType.DMA,
        pltpu.SemaphoreType.DMA,
        pltpu.SemaphoreType.DMA,
    ],
)
def _edge_pass_b(pack_hbm, ex_hbm, den_hbm, z_hbm, part_hbm,
                 packA, packB, exA, exB, alA, alB, dstA, dstB, den_v,
                 rowsA, rowsB, accum_sh,
                 spA, spB, seA, seB, sgA, sgB, ssA, ssB):
    c = lax.axis_index("c")
    s = lax.axis_index("s")
    w = s * NC + c
    pltpu.sync_copy(den_hbm, den_v)

    def zb(i, carry):
        r = i // 8
        kk = i % 8
        rowsA[r, pl.ds(kk * 16, 16)] = jnp.zeros((16,), jnp.float32)
        return carry

    lax.fori_loop(0, CLB * 8, zb, 0)
    for t in range(RPT // CLB):
        pltpu.sync_copy(rowsA, accum_sh.at[pl.ds(s * RPT + t * CLB, CLB)])
    plsc.subcore_barrier()

    pltpu.async_copy(pack_hbm.at[w, 0], packA, spA)
    pltpu.async_copy(ex_hbm.at[w, 0], exA, seA)
    pltpu.async_copy(pack_hbm.at[w, 1], packB, spB)
    pltpu.async_copy(ex_hbm.at[w, 1], exB, seB)

    def _alpha(pack, ex, al, dst):
        for k in range(CLB // 16):
            sl = pl.ds(k * 16, 16)
            dv = pack[1, sl]
            dst[sl] = dv
            den = plsc.load_gather(den_v, [dv])
            al[sl] = ex[sl] / jnp.maximum(den, 1e-38)

    def _scale(rows, al):
        @plsc.parallel_loop(0, CLB, 1, unroll=2)
        def _(e):
            alv = plsc.load_gather(al, [jnp.full((16,), e, jnp.int32)])
            for kk in range(D // 16):
                sl = pl.ds(kk * 16, 16)
                rows[e, sl] = rows[e, sl] * alv

    def body(i, carry):
        jA = 2 * i
        jB = 2 * i + 1
        pltpu.make_async_copy(pack_hbm.at[w, jA], packA, spA).wait()
        pltpu.make_async_copy(ex_hbm.at[w, jA], exA, seA).wait()

        @pl.when(i > 0)
        def _():
            pltpu.make_async_copy(rowsA, accum_sh.at[dstA], ssA).wait()

        pltpu.async_copy(z_hbm.at[packA.at[0]], rowsA, sgA)
        _alpha(packA, exA, alA, dstA)

        pltpu.make_async_copy(pack_hbm.at[w, jB], packB, spB).wait()
        pltpu.make_async_copy(ex_hbm.at[w, jB], exB, seB).wait()

        @pl.when(i > 0)
        def _():
            pltpu.make_async_copy(rowsB, accum_sh.at[dstB], ssB).wait()

        pltpu.async_copy(z_hbm.at[packB.at[0]], rowsB, sgB)
        _alpha(packB, exB, alB, dstB)

        pltpu.make_async_copy(z_hbm.at[packA.at[0]], rowsA, sgA).wait()

        @pl.when(i < HALF - 1)
        def _():
            pltpu.async_copy(pack_hbm.at[w, jA + 2], packA, spA)
            pltpu.async_copy(ex_hbm.at[w, jA + 2], exA, seA)

        _scale(rowsA, alA)
        pltpu.async_copy(rowsA, accum_sh.at[dstA], ssA, add=True)

        pltpu.make_async_copy(z_hbm.at[packB.at[0]], rowsB, sgB).wait()

        @pl.when(i < HALF - 1)
        def _():
            pltpu.async_copy(pack_hbm.at[w, jB + 2], packB, spB)
            pltpu.async_copy(ex_hbm.at[w, jB + 2], exB, seB)

        _scale(rowsB, alB)
        pltpu.async_copy(rowsB, accum_sh.at[dstB], ssB, add=True)
        return carry

    lax.fori_loop(0, HALF, body, 0)
    pltpu.make_async_copy(rowsA, accum_sh.at[dstA], ssA).wait()
    pltpu.make_async_copy(rowsB, accum_sh.at[dstB], ssB).wait()
    plsc.subcore_barrier()
    for t in range(RPT // 128):
        r0 = s * RPT + t * 128
        pltpu.sync_copy(accum_sh.at[pl.ds(r0, 128)],
                        part_hbm.at[c, pl.ds(r0, 128)])


def kernel(h, edge_index, W_fc, b_fc, W_a, b_a):
    z, a = _fused_matmul(h, W_fc, b_fc, W_a, b_a)
    a12 = a.T
    pad = EP - E
    src_p = jnp.concatenate(
        [edge_index[0].astype(jnp.int32), jnp.zeros((pad,), jnp.int32)]
    ).reshape(NW, CHUNKS, CL)
    dst_p = jnp.concatenate(
        [edge_index[1].astype(jnp.int32), jnp.zeros((pad,), jnp.int32)]
    ).reshape(NW, CHUNKS, CL)
    ex, den = _edge_pass_a(src_p, dst_p, a12)
    pack = jnp.stack(
        [src_p.reshape(NW, NCHB, CLB), dst_p.reshape(NW, NCHB, CLB)], axis=2
    )
    part = _edge_pass_b(pack, ex.reshape(NW, NCHB, CLB), _den_add(den), z)
    return _combine(part[0], part[1])


# speedup vs baseline: 20.1951x; 1.4859x over previous
import functools

import jax
import jax.numpy as jnp
from jax import lax
from jax.experimental import pallas as pl
from jax.experimental.pallas import tpu as pltpu
from jax.experimental.pallas import tpu_sc as plsc

N = 10000
NP = 10240
D = 128
E = 320000
NC = 2
NS = 16
NW = NC * NS
EPW = 10240
CHUNKS = 80
CL = 128
EP = NW * EPW
RPT = NP // NS
ROW_BLK = 2000


def _matmul_body(h_ref, wt_ref, b_ref, wa_ref, ba_ref, z_ref, a_ref):
    z = jnp.dot(h_ref[...], wt_ref[...], preferred_element_type=jnp.float32)
    z = z + b_ref[...]
    z_ref[...] = z.astype(jnp.bfloat16)
    a_ref[...] = (
        jnp.dot(z, wa_ref[...], preferred_element_type=jnp.float32) + ba_ref[...]
    )


def _fused_matmul(h, W_fc, b_fc, W_a, b_a):
    idx = jnp.arange(D)
    inv = 32 * (idx // 32) + (idx % 32) // 2 + 16 * (idx % 2)
    wt = W_fc.T[:, inv]
    bp = b_fc[inv]
    wa = W_a.reshape(2, D).T[inv, :]
    ba2 = jnp.concatenate([jnp.zeros((1,), jnp.float32), b_a])
    grid = N // ROW_BLK
    z, a = pl.pallas_call(
        _matmul_body,
        grid=(grid,),
        in_specs=[
            pl.BlockSpec((ROW_BLK, D), lambda i: (i, 0)),
            pl.BlockSpec((D, D), lambda i: (0, 0)),
            pl.BlockSpec((D,), lambda i: (0,)),
            pl.BlockSpec((D, 2), lambda i: (0, 0)),
            pl.BlockSpec((2,), lambda i: (0,)),
        ],
        out_specs=[
            pl.BlockSpec((ROW_BLK, D), lambda i: (i, 0)),
            pl.BlockSpec((ROW_BLK, 2), lambda i: (i, 0)),
        ],
        out_shape=[
            jax.ShapeDtypeStruct((NP, D), jnp.bfloat16),
            jax.ShapeDtypeStruct((NP, 2), jnp.float32),
        ],
    )(h, wt, bp, wa, ba2)
    return z, a


def _combine_body(p0_ref, p1_ref, o_ref):
    o_ref[...] = p0_ref[...] + p1_ref[...]


def _den_add_body(d_ref, o_ref):
    o_ref[...] = d_ref[0] + d_ref[1]


def _den_add(den):
    out = pl.pallas_call(
        _den_add_body,
        out_shape=jax.ShapeDtypeStruct((NP // D, D), jnp.float32),
    )(den.reshape(NC, NP // D, D))
    return out.reshape(NP)


def _combine(p0, p1):
    return pl.pallas_call(
        _combine_body,
        grid=(N // ROW_BLK,),
        in_specs=[
            pl.BlockSpec((ROW_BLK, D), lambda i: (i, 0)),
            pl.BlockSpec((ROW_BLK, D), lambda i: (i, 0)),
        ],
        out_specs=pl.BlockSpec((ROW_BLK, D), lambda i: (i, 0)),
        out_shape=jax.ShapeDtypeStruct((N, D), jnp.float32),
    )(p0, p1)


_MESH = plsc.VectorSubcoreMesh(core_axis_name="c", subcore_axis_name="s")
_SC_PARAMS = pltpu.CompilerParams(
    needs_layout_passes=False, use_tc_tiling_on_sc=False
)


@functools.partial(
    pl.kernel,
    out_type=[
        jax.ShapeDtypeStruct((NW, EPW), jnp.float32),
        jax.ShapeDtypeStruct((NC, NP), jnp.float32),
    ],
    mesh=_MESH,
    compiler_params=_SC_PARAMS,
    scratch_types=[
        pltpu.VMEM((NP,), jnp.float32),
        pltpu.VMEM((NP,), jnp.float32),
        pltpu.VMEM((CHUNKS, CL), jnp.int32),
        pltpu.VMEM((CHUNKS, CL), jnp.int32),
        pltpu.VMEM((EPW,), jnp.float32),
        pltpu.VMEM((RPT,), jnp.float32),
        pltpu.VMEM_SHARED((NP,), jnp.float32),
    ],
)
def _edge_pass_a(src_hbm, dst_hbm, a12_hbm, ex_hbm, den_hbm,
                 a1_v, a2_v, src_v, dst_v, ex_v, zero_v, den_sh):
    c = lax.axis_index("c")
    s = lax.axis_index("s")
    w = s * NC + c
    pltpu.sync_copy(a12_hbm.at[0], a1_v)
    pltpu.sync_copy(a12_hbm.at[1], a2_v)
    pltpu.sync_copy(src_hbm.at[w], src_v)
    pltpu.sync_copy(dst_hbm.at[w], dst_v)

    def zb(i, carry):
        zero_v[pl.ds(i * 16, 16)] = jnp.zeros((16,), jnp.float32)
        return carry

    lax.fori_loop(0, RPT // 16, zb, 0)
    pltpu.sync_copy(zero_v, den_sh.at[pl.ds(s * RPT, RPT)])
    plsc.subcore_barrier()

    iota16 = lax.iota(jnp.int32, 16)
    wbase = w * EPW

    def chunk_body(j, carry):
        for k in range(CL // 16):
            srcv = src_v[j, pl.ds(k * 16, 16)]
            dstv = dst_v[j, pl.ds(k * 16, 16)]
            a1 = plsc.load_gather(a1_v, [srcv])
            a2 = plsc.load_gather(a2_v, [dstv])
            e = a1 + a2
            e = jnp.where(e >= 0.0, e, 0.01 * e)
            flat = wbase + j * CL + k * 16 + iota16
            ex = jnp.where(flat < E, jnp.exp(e), 0.0)
            ex_v[pl.ds(j * CL + k * 16, 16)] = ex
        pltpu.sync_copy(ex_v.at[pl.ds(j * CL, CL)],
                        den_sh.at[dst_v.at[j]], add=True)
        return carry

    lax.fori_loop(0, CHUNKS, chunk_body, 0)
    plsc.subcore_barrier()
    pltpu.sync_copy(ex_v, ex_hbm.at[w])
    pltpu.sync_copy(den_sh.at[pl.ds(s * RPT, RPT)],
                    den_hbm.at[c, pl.ds(s * RPT, RPT)])


CLB = 64
NCHB = EPW // CLB
HALF = NCHB // 2


@functools.partial(
    pl.kernel,
    out_type=jax.ShapeDtypeStruct((NC, NP, D), jnp.float32),
    mesh=_MESH,
    compiler_params=_SC_PARAMS,
    scratch_types=[
        pltpu.VMEM((2, CLB), jnp.int32),
        pltpu.VMEM((2, CLB), jnp.int32),
        pltpu.VMEM((CLB,), jnp.float32),
        pltpu.VMEM((CLB,), jnp.float32),
        pltpu.VMEM((CLB,), jnp.float32),
        pltpu.VMEM((CLB,), jnp.float32),
        pltpu.VMEM((CLB,), jnp.int32),
        pltpu.VMEM((CLB,), jnp.int32),
        pltpu.VMEM((NP,), jnp.float32),
        pltpu.VMEM((CLB, D // 2), jnp.int32),
        pltpu.VMEM((CLB, D // 2), jnp.int32),
        pltpu.VMEM((CLB, D), jnp.float32),
        pltpu.VMEM((CLB, D), jnp.float32),
        pltpu.VMEM_SHARED((NP, D), jnp.float32),
        pltpu.SemaphoreType.DMA,
        pltpu.SemaphoreType.DMA,
        pltpu.SemaphoreType.DMA,
        pltpu.SemaphoreType.DMA,
        pltpu.SemaphoreType.DMA,
        pltpu.SemaphoreType.DMA,
        pltpu.SemaphoreType.DMA,
        pltpu.SemaphoreType.DMA,
    ],
)
def _edge_pass_b(pack_hbm, ex_hbm, den_hbm, z_hbm, part_hbm,
                 packA, packB, exA, exB, alA, alB, dstA, dstB, den_v,
                 rowsA, rowsB, frowsA, frowsB, accum_sh,
                 spA, spB, seA, seB, sgA, sgB, ssA, ssB):
    c = lax.axis_index("c")
    s = lax.axis_index("s")
    w = s * NC + c
    pltpu.sync_copy(den_hbm, den_v)

    def zb(i, carry):
        r = i // 8
        kk = i % 8
        frowsA[r, pl.ds(kk * 16, 16)] = jnp.zeros((16,), jnp.float32)
        return carry

    lax.fori_loop(0, CLB * 8, zb, 0)
    for t in range(RPT // CLB):
        pltpu.sync_copy(frowsA, accum_sh.at[pl.ds(s * RPT + t * CLB, CLB)])
    plsc.subcore_barrier()

    pltpu.async_copy(pack_hbm.at[w, 0], packA, spA)
    pltpu.async_copy(ex_hbm.at[w, 0], exA, seA)
    pltpu.async_copy(pack_hbm.at[w, 1], packB, spB)
    pltpu.async_copy(ex_hbm.at[w, 1], exB, seB)

    def _alpha(pack, ex, al, dst):
        for k in range(CLB // 16):
            sl = pl.ds(k * 16, 16)
            dv = pack[1, sl]
            dst[sl] = dv
            den = plsc.load_gather(den_v, [dv])
            al[sl] = ex[sl] / jnp.maximum(den, 1e-38)

    _HI = jnp.full((16,), -65536, jnp.int32)

    def _scale(rows, frows, al):
        @plsc.parallel_loop(0, CLB, 1, unroll=2)
        def _(e):
            alv = plsc.load_gather(al, [jnp.full((16,), e, jnp.int32)])
            for kk in range(D // 32):
                v = rows[e, pl.ds(kk * 16, 16)]
                lo = plsc.bitcast(v << 16, jnp.float32)
                hi = plsc.bitcast(v & _HI, jnp.float32)
                frows[e, pl.ds(kk * 32, 16)] = lo * alv
                frows[e, pl.ds(kk * 32 + 16, 16)] = hi * alv

    def body(i, carry):
        jA = 2 * i
        jB = 2 * i + 1
        pltpu.make_async_copy(pack_hbm.at[w, jA], packA, spA).wait()
        pltpu.make_async_copy(ex_hbm.at[w, jA], exA, seA).wait()

        @pl.when(i > 0)
        def _():
            pltpu.make_async_copy(frowsA, accum_sh.at[dstA], ssA).wait()

        pltpu.async_copy(z_hbm.at[packA.at[0]], rowsA, sgA)
        _alpha(packA, exA, alA, dstA)

        pltpu.make_async_copy(pack_hbm.at[w, jB], packB, spB).wait()
        pltpu.make_async_copy(ex_hbm.at[w, jB], exB, seB).wait()

        @pl.when(i > 0)
        def _():
            pltpu.make_async_copy(frowsB, accum_sh.at[dstB], ssB).wait()

        pltpu.async_copy(z_hbm.at[packB.at[0]], rowsB, sgB)
        _alpha(packB, exB, alB, dstB)

        pltpu.make_async_copy(z_hbm.at[packA.at[0]], rowsA, sgA).wait()

        @pl.when(i < HALF - 1)
        def _():
            pltpu.async_copy(pack_hbm.at[w, jA + 2], packA, spA)
            pltpu.async_copy(ex_hbm.at[w, jA + 2], exA, seA)

        _scale(rowsA, frowsA, alA)
        pltpu.async_copy(frowsA, accum_sh.at[dstA], ssA, add=True)

        pltpu.make_async_copy(z_hbm.at[packB.at[0]], rowsB, sgB).wait()

        @pl.when(i < HALF - 1)
        def _():
            pltpu.async_copy(pack_hbm.at[w, jB + 2], packB, spB)
            pltpu.async_copy(ex_hbm.at[w, jB + 2], exB, seB)

        _scale(rowsB, frowsB, alB)
        pltpu.async_copy(frowsB, accum_sh.at[dstB], ssB, add=True)
        return carry

    lax.fori_loop(0, HALF, body, 0)
    pltpu.make_async_copy(frowsA, accum_sh.at[dstA], ssA).wait()
    pltpu.make_async_copy(frowsB, accum_sh.at[dstB], ssB).wait()
    plsc.subcore_barrier()
    for t in range(RPT // 128):
        r0 = s * RPT + t * 128
        pltpu.sync_copy(accum_sh.at[pl.ds(r0, 128)],
                        part_hbm.at[c, pl.ds(r0, 128)])


def kernel(h, edge_index, W_fc, b_fc, W_a, b_a):
    z, a = _fused_matmul(h, W_fc, b_fc, W_a, b_a)
    a12 = a.T
    pad = EP - E
    src_p = jnp.concatenate(
        [edge_index[0].astype(jnp.int32), jnp.zeros((pad,), jnp.int32)]
    ).reshape(NW, CHUNKS, CL)
    dst_p = jnp.concatenate(
        [edge_index[1].astype(jnp.int32), jnp.zeros((pad,), jnp.int32)]
    ).reshape(NW, CHUNKS, CL)
    ex, den = _edge_pass_a(src_p, dst_p, a12)
    pack = jnp.stack(
        [src_p.reshape(NW, NCHB, CLB), dst_p.reshape(NW, NCHB, CLB)], axis=2
    )
    z_pack = jax.lax.bitcast_convert_type(
        z.reshape(NP, D // 2, 2), jnp.int32
    )
    part = _edge_pass_b(pack, ex.reshape(NW, NCHB, CLB), _den_add(den), z_pack)
    return _combine(part[0], part[1])
